# Initial kernel scaffold; baseline (speedup 1.0000x reference)
import jax
import jax.numpy as jnp
from jax.experimental import pallas as pl


def kernel(xyz):
    raise NotImplementedError("write your pallas kernel here")


# speedup vs baseline: 11.5517x; 11.5517x over previous
import jax
import jax.numpy as jnp
from jax.experimental import pallas as pl
from jax.experimental.pallas import tpu as pltpu

_B, _N, _D = 16, 4096, 3
_G, _K = 256, 32
_BIG = 1e30


def _fps_body(x_ref, y_ref, z_ref, cidx_ref, cx_ref, cy_ref, cz_ref, dist_ref):
    x = x_ref[0]
    y = y_ref[0]
    z = z_ref[0]
    iota_n = jax.lax.broadcasted_iota(jnp.int32, (_B, _N), 1)
    iota_g = jax.lax.broadcasted_iota(jnp.int32, (_B, _G), 1)
    dist_ref[...] = jnp.full((_B, _N), 1e10, jnp.float32)
    cidx_ref[...] = jnp.zeros((_B, _G), jnp.int32)
    cx_ref[...] = jnp.zeros((_B, _G), jnp.float32)
    cy_ref[...] = jnp.zeros((_B, _G), jnp.float32)
    cz_ref[...] = jnp.zeros((_B, _G), jnp.float32)

    def body(i, carry):
        dist = dist_ref[...]
        m = jnp.max(dist, axis=1, keepdims=True)
        far = jnp.min(jnp.where(dist == m, iota_n, _N), axis=1, keepdims=True)
        oh_i = (iota_g == i).astype(jnp.int32)
        oh_f = oh_i.astype(jnp.float32)
        cidx_ref[...] = cidx_ref[...] + oh_i * far
        sel = iota_n == far
        fx = jnp.sum(jnp.where(sel, x, 0.0), axis=1, keepdims=True)
        fy = jnp.sum(jnp.where(sel, y, 0.0), axis=1, keepdims=True)
        fz = jnp.sum(jnp.where(sel, z, 0.0), axis=1, keepdims=True)
        cx_ref[...] = cx_ref[...] + oh_f * fx
        cy_ref[...] = cy_ref[...] + oh_f * fy
        cz_ref[...] = cz_ref[...] + oh_f * fz
        dx = x - fx
        dy = y - fy
        dz = z - fz
        d = (dx * dx + dy * dy) + dz * dz
        dist_ref[...] = jnp.minimum(dist, d)
        return carry

    jax.lax.fori_loop(0, _G, body, 0)


def _knn_body(x_ref, y_ref, z_ref, cx_ref, cy_ref, cz_ref, c3_ref, p3t_ref,
              idx_ref, d2_ref):
    x = x_ref[0]
    y = y_ref[0]
    z = z_ref[0]
    cx = cx_ref[0]
    cy = cy_ref[0]
    cz = cz_ref[0]
    psq = (x * x + y * y) + z * z
    csq = (cx * cx + cy * cy) + cz * cz
    dot = jax.lax.dot_general(
        c3_ref[0], p3t_ref[0], (((1,), (0,)), ((), ())),
        precision=jax.lax.Precision.DEFAULT,
        preferred_element_type=jnp.float32)
    d2_ref[...] = (csq + psq) - 2.0 * dot
    iota_n = jax.lax.broadcasted_iota(jnp.int32, (_G, _N), 1)
    for k in range(_K):
        d2 = d2_ref[...]
        m = jnp.min(d2, axis=1, keepdims=True)
        sel = jnp.min(jnp.where(d2 == m, iota_n, _N), axis=1, keepdims=True)
        idx_ref[0, :, pl.ds(k, 1)] = sel
        d2_ref[...] = jnp.where(iota_n == sel, _BIG, d2)


def kernel(xyz):
    xt = jnp.transpose(xyz, (2, 0, 1))
    x3 = xt[:, None]
    x = x3[0]
    y = x3[1]
    z = x3[2]

    fps = pl.pallas_call(
        _fps_body,
        grid=(1,),
        in_specs=[pl.BlockSpec((1, _B, _N), lambda i: (0, 0, 0))] * 3,
        out_specs=[pl.BlockSpec((_B, _G), lambda i: (0, 0))] * 4,
        out_shape=[
            jax.ShapeDtypeStruct((_B, _G), jnp.int32),
            jax.ShapeDtypeStruct((_B, _G), jnp.float32),
            jax.ShapeDtypeStruct((_B, _G), jnp.float32),
            jax.ShapeDtypeStruct((_B, _G), jnp.float32),
        ],
        scratch_shapes=[pltpu.VMEM((_B, _N), jnp.float32)],
    )
    cidx, cx, cy, cz = fps(x, y, z)

    knn = pl.pallas_call(
        _knn_body,
        grid=(_B,),
        in_specs=[
            pl.BlockSpec((1, 1, _N), lambda i: (i, 0, 0)),
            pl.BlockSpec((1, 1, _N), lambda i: (i, 0, 0)),
            pl.BlockSpec((1, 1, _N), lambda i: (i, 0, 0)),
            pl.BlockSpec((1, _G, 1), lambda i: (i, 0, 0)),
            pl.BlockSpec((1, _G, 1), lambda i: (i, 0, 0)),
            pl.BlockSpec((1, _G, 1), lambda i: (i, 0, 0)),
            pl.BlockSpec((1, _G, _D), lambda i: (i, 0, 0)),
            pl.BlockSpec((1, _D, _N), lambda i: (i, 0, 0)),
        ],
        out_specs=pl.BlockSpec((1, _G, _K), lambda i: (i, 0, 0)),
        out_shape=jax.ShapeDtypeStruct((_B, _G, _K), jnp.int32),
        scratch_shapes=[pltpu.VMEM((_G, _N), jnp.float32)],
    )
    center = jnp.stack([cx, cy, cz], axis=-1)
    p3t = jnp.transpose(xyz, (0, 2, 1))
    idx = knn(
        x.reshape(_B, 1, _N), y.reshape(_B, 1, _N), z.reshape(_B, 1, _N),
        cx[:, :, None], cy[:, :, None], cz[:, :, None],
        center, p3t,
    )
    return (idx, cidx, center)
